# baseline (speedup 1.0000x reference)
import jax
import jax.numpy as jnp
from jax import lax
from jax.experimental import pallas as pl
from jax.experimental.pallas import tpu as pltpu
from jax.experimental.pallas import tpu_sc as plsc

N_NODES = 10000
H = 256
N_RELS = 200
N_TRIP = 160000
NC = 2
NS = 16
NW = NC * NS
PER_W = N_TRIP // NW
B = 40
NCH = PER_W // B
NCHG = N_TRIP // B
VPT = H // 16

_DNUMS = lax.GatherDimensionNumbers(
    offset_dims=(), collapsed_slice_dims=(0,), start_index_map=(0,))


def _body(idx_hbm, emb_hbm, wrel_hbm, out_hbm,
          w_v, s_v, o_v, idx_v, out_v, sem_s, sem_o, sem_i):
    cid = lax.axis_index("c")
    sid = lax.axis_index("s")
    wid = sid * NC + cid
    gc0 = wid * NCH

    pltpu.sync_copy(wrel_hbm, w_v)

    iota = lax.iota(jnp.int32, 16)
    m15 = iota == 15

    def idx_copy(c, ib):
        return pltpu.make_async_copy(
            idx_hbm.at[gc0 + c], idx_v.at[ib], sem_i.at[ib])

    def gather_s(b, ib):
        return pltpu.make_async_copy(
            emb_hbm.at[idx_v.at[ib, pl.ds(0, B)]], s_v.at[b], sem_s.at[b])

    def gather_o(b, ib):
        return pltpu.make_async_copy(
            emb_hbm.at[idx_v.at[ib, pl.ds(2 * B, B)]], o_v.at[b], sem_o.at[b])

    def gather_start(b, ib):
        gather_s(b, ib).start()
        gather_o(b, ib).start()

    def gather_wait(b, ib):
        gather_s(b, ib).wait()
        gather_o(b, ib).wait()

    def compute(c, b, ib):
        out_base = c * B
        for g in range(3):
            cnt = 16 if g < 2 else B - 32
            rv = idx_v[ib, pl.ds(B + 16 * g, 16)]
            rr = lax.rem(rv, jnp.int32(N_RELS)) * (H // 2)

            def tri(i, carry):
                ti = g * 16 + i
                rri = lax.gather(
                    rr, (jnp.full((16,), 0, jnp.int32) + i)[:, None],
                    _DNUMS, (1,), mode=lax.GatherScatterMode.PROMISE_IN_BOUNDS)
                acc = None
                for j in range(VPT // 2):
                    sv2 = s_v[b, ti, pl.ds(32 * j, 32)]
                    ov2 = o_v[b, ti, pl.ds(32 * j, 32)]
                    po = sv2 * ov2
                    pa, pb = plsc.unpack(po, format=plsc.PackFormat.INTERLEAVED)
                    rp = plsc.load_gather(w_v, [rri + (iota + 16 * j)])
                    rbf = plsc.bitcast(rp, jnp.bfloat16)
                    ra, rb = plsc.unpack(rbf, format=plsc.PackFormat.INTERLEAVED)
                    term = pa * ra + pb * rb
                    acc = term if acc is None else acc + term
                cum = jnp.cumsum(acc)
                pos = out_base + ti
                plsc.store_scatter(out_v, [jnp.full((16,), 0, jnp.int32) + pos],
                                   cum, mask=m15)
                return carry

            lax.fori_loop(0, cnt, tri, 0, unroll=2)

    idx_copy(0, 0).start()
    idx_copy(1, 1).start()
    idx_copy(0, 0).wait()
    idx_copy(1, 1).wait()
    idx_copy(2, 2).start()
    idx_copy(3, 3).start()
    gather_start(0, 0)
    gather_start(1, 1)

    def step(t, carry):
        c = t * 4
        for k in range(4):
            x = c + k
            idx_copy(x + 2, (k + 2) % 4).wait()
            gather_start((k + 2) % 4, (k + 2) % 4)
            gather_wait(k % 4, k % 4)
            compute(x, k % 4, k % 4)
            idx_copy(x + 4, k % 4).start()
        return carry

    lax.fori_loop(0, (NCH - 1) // 4, step, 0, unroll=False)
    gather_wait(0, 0)
    compute(NCH - 1, 0, 0)
    gather_wait(1, 1)
    pltpu.sync_copy(out_v, out_hbm.at[pl.ds(wid * PER_W, PER_W)])


def kernel(embedding, w_relation, triplets):
    idx_blocks = triplets.reshape(NCHG, B, 3).transpose(0, 2, 1).reshape(NCHG, 3 * B)
    idx_blocks = jnp.pad(idx_blocks, ((0, 8), (0, 0)))
    mesh = plsc.VectorSubcoreMesh(core_axis_name="c", subcore_axis_name="s",
                                  num_cores=NC, num_subcores=NS)
    f = pl.kernel(
        _body,
        out_type=jax.ShapeDtypeStruct((N_TRIP,), jnp.float32),
        mesh=mesh,
        compiler_params=pltpu.CompilerParams(use_tc_tiling_on_sc=False,
                                             needs_layout_passes=False),
        scratch_types=[
            pltpu.VMEM((N_RELS * H // 2,), jnp.int32),
            pltpu.VMEM((4, B, H), jnp.bfloat16),
            pltpu.VMEM((4, B, H), jnp.bfloat16),
            pltpu.VMEM((4, 3 * B), jnp.int32),
            pltpu.VMEM((PER_W,), jnp.float32),
            pltpu.SemaphoreType.DMA((4,)),
            pltpu.SemaphoreType.DMA((4,)),
            pltpu.SemaphoreType.DMA((4,)),
        ],
    )
    w_pack = jax.lax.bitcast_convert_type(
        w_relation.astype(jnp.bfloat16).reshape(N_RELS * H // 2, 2), jnp.int32)
    return f(idx_blocks, embedding.astype(jnp.bfloat16), w_pack)


# speedup vs baseline: 1.2968x; 1.2968x over previous
import jax
import jax.numpy as jnp
from jax import lax
from jax.experimental import pallas as pl
from jax.experimental.pallas import tpu as pltpu
from jax.experimental.pallas import tpu_sc as plsc

N_NODES = 10000
H = 256
N_RELS = 200
N_TRIP = 160000
NC = 2
NS = 16
NW = NC * NS
PER_W = N_TRIP // NW
B = 40
NCH = PER_W // B
NCHG = N_TRIP // B
VPT = H // 16

_DNUMS = lax.GatherDimensionNumbers(
    offset_dims=(), collapsed_slice_dims=(0,), start_index_map=(0,))


def _body(idx_hbm, emb_hbm, wrel_hbm, out_hbm,
          w_v, s_v, o_v, idx_v, out_v, sem_s, sem_o, sem_i):
    cid = lax.axis_index("c")
    sid = lax.axis_index("s")
    wid = sid * NC + cid
    gc0 = wid * NCH

    pltpu.sync_copy(wrel_hbm, w_v)

    iota = lax.iota(jnp.int32, 16)
    m15 = iota == 15

    def idx_copy(c, ib):
        return pltpu.make_async_copy(
            idx_hbm.at[gc0 + c], idx_v.at[ib], sem_i.at[ib])

    def gather_s(b, ib):
        return pltpu.make_async_copy(
            emb_hbm.at[idx_v.at[ib, pl.ds(0, B)]], s_v.at[b], sem_s.at[b])

    def gather_o(b, ib):
        return pltpu.make_async_copy(
            emb_hbm.at[idx_v.at[ib, pl.ds(2 * B, B)]], o_v.at[b], sem_o.at[b])

    def gather_start(b, ib):
        gather_s(b, ib).start()
        gather_o(b, ib).start()

    def gather_wait(b, ib):
        gather_s(b, ib).wait()
        gather_o(b, ib).wait()

    def compute(c, b, ib):
        out_base = c * B
        for g in range(3):
            cnt = 16 if g < 2 else B - 32
            rv = idx_v[ib, pl.ds(B + 16 * g, 16)]
            rr = lax.rem(rv, jnp.int32(N_RELS)) * (H // 2)

            def tri(i, carry):
                ti = g * 16 + i
                rri = lax.gather(
                    rr, (jnp.full((16,), 0, jnp.int32) + i)[:, None],
                    _DNUMS, (1,), mode=lax.GatherScatterMode.PROMISE_IN_BOUNDS)
                acc = None
                for j in range(VPT // 2):
                    sv2 = s_v[b, ti, pl.ds(32 * j, 32)]
                    ov2 = o_v[b, ti, pl.ds(32 * j, 32)]
                    po = sv2 * ov2
                    pa, pb = plsc.unpack(po, format=plsc.PackFormat.INTERLEAVED)
                    rp = plsc.load_gather(w_v, [rri + (iota + 16 * j)])
                    rbf = plsc.bitcast(rp, jnp.bfloat16)
                    ra, rb = plsc.unpack(rbf, format=plsc.PackFormat.INTERLEAVED)
                    term = pa * ra + pb * rb
                    acc = term if acc is None else acc + term
                cum = jnp.cumsum(acc)
                pos = out_base + ti
                plsc.store_scatter(out_v, [jnp.full((16,), 0, jnp.int32) + pos],
                                   cum, mask=m15)
                return carry

            lax.fori_loop(0, cnt, tri, 0, unroll=False)

    idx_copy(0, 0).start()
    idx_copy(1, 1).start()
    idx_copy(0, 0).wait()
    idx_copy(1, 1).wait()
    idx_copy(2, 2).start()
    idx_copy(3, 3).start()
    gather_start(0, 0)
    gather_start(1, 1)

    def step(c, carry):
        b4 = lax.rem(c, jnp.int32(4))
        n4 = lax.rem(c + 2, jnp.int32(4))
        idx_copy(c + 2, n4).wait()
        gather_start(n4, n4)
        gather_wait(b4, b4)
        compute(c, b4, b4)
        idx_copy(c + 4, b4).start()
        return carry

    lax.fori_loop(0, NCH - 1, step, 0, unroll=False)
    gather_wait(0, 0)
    compute(NCH - 1, 0, 0)
    gather_wait(1, 1)
    pltpu.sync_copy(out_v, out_hbm.at[pl.ds(wid * PER_W, PER_W)])


def kernel(embedding, w_relation, triplets):
    idx_blocks = triplets.reshape(NCHG, B, 3).transpose(0, 2, 1).reshape(NCHG, 3 * B)
    idx_blocks = jnp.pad(idx_blocks, ((0, 8), (0, 0)))
    mesh = plsc.VectorSubcoreMesh(core_axis_name="c", subcore_axis_name="s",
                                  num_cores=NC, num_subcores=NS)
    f = pl.kernel(
        _body,
        out_type=jax.ShapeDtypeStruct((N_TRIP,), jnp.float32),
        mesh=mesh,
        compiler_params=pltpu.CompilerParams(use_tc_tiling_on_sc=False,
                                             needs_layout_passes=False),
        scratch_types=[
            pltpu.VMEM((N_RELS * H // 2,), jnp.int32),
            pltpu.VMEM((4, B, H), jnp.bfloat16),
            pltpu.VMEM((4, B, H), jnp.bfloat16),
            pltpu.VMEM((4, 3 * B), jnp.int32),
            pltpu.VMEM((PER_W,), jnp.float32),
            pltpu.SemaphoreType.DMA((4,)),
            pltpu.SemaphoreType.DMA((4,)),
            pltpu.SemaphoreType.DMA((4,)),
        ],
    )
    w_pack = jax.lax.bitcast_convert_type(
        w_relation.astype(jnp.bfloat16).reshape(N_RELS * H // 2, 2), jnp.int32)
    return f(idx_blocks, embedding.astype(jnp.bfloat16), w_pack)
